# baseline (speedup 1.0000x reference)
import functools

import jax
import jax.numpy as jnp
from jax.experimental import pallas as pl
from jax.experimental.pallas import tpu as pltpu


def _tree(vals, op):
    while len(vals) > 1:
        nxt = [op(vals[i], vals[i + 1]) for i in range(0, len(vals) - 1, 2)]
        if len(vals) % 2:
            nxt.append(vals[-1])
        vals = nxt
    return vals[0]


def _radix_pass(ukey, prefix, cnt_at, sh, m, kf, batch):
    inds, cnts = [], []
    for j in range(1, 1 << m):
        cand = prefix | jax.lax.shift_left(jnp.uint32(j), sh)
        cnt = jnp.sum(jnp.where(ukey >= cand, 1.0, 0.0), axis=1,
                      keepdims=True)
        ok = cnt >= kf
        inds.append(jnp.where(ok, jnp.uint32(1), jnp.uint32(0)))
        cnts.append(jnp.where(ok, cnt, jnp.float32(3e9)))
    jstar = _tree(inds, lambda a, b: a + b)
    best = _tree(cnts, jnp.minimum)
    prefix = prefix | jax.lax.shift_left(jstar, sh)
    cnt_at = jnp.minimum(cnt_at, best)
    return prefix, cnt_at


def _kth_threshold(ukey, kf, batch, rec):
    prefix = jnp.zeros((batch, 1), jnp.uint32)
    cnt_at = jnp.full((batch, 1), float(rec), jnp.float32)
    prefix, cnt_at = _radix_pass(ukey, prefix, cnt_at, jnp.uint32(30), 2, kf,
                                 batch)
    prefix, cnt_at = _radix_pass(ukey, prefix, cnt_at, jnp.uint32(27), 3, kf,
                                 batch)
    prefix, cnt_at = _radix_pass(ukey, prefix, cnt_at, jnp.uint32(24), 3, kf,
                                 batch)
    prefix, cnt_at = _radix_pass(ukey, prefix, cnt_at, jnp.uint32(21), 3, kf,
                                 batch)
    prefix, cnt_at = _radix_pass(ukey, prefix, cnt_at, jnp.uint32(18), 3, kf,
                                 batch)
    prefix, cnt_at = _radix_pass(ukey, prefix, cnt_at, jnp.uint32(15), 3, kf,
                                 batch)
    prefix, cnt_at = _radix_pass(ukey, prefix, cnt_at, jnp.uint32(12), 3, kf,
                                 batch)
    prefix, cnt_at = _radix_pass(ukey, prefix, cnt_at, jnp.uint32(9), 3, kf,
                                 batch)

    def cond_fn(c):
        i, _, cnt_at = c
        return jnp.logical_and(i < 3, jnp.any(cnt_at != kf))

    def body_fn(c):
        i, prefix, cnt_at = c
        sh = jnp.uint32(6) - jnp.uint32(3) * i.astype(jnp.uint32)
        prefix, cnt_at = _radix_pass(ukey, prefix, cnt_at, sh, 3, kf, batch)
        return i + 1, prefix, cnt_at

    _, prefix, _ = jax.lax.while_loop(
        cond_fn, body_fn, (jnp.int32(0), prefix, cnt_at))
    return prefix


def _encoder_kernel(x_ref, wi_ref, wr_ref, out_ref, u_ref, *, steps, batch,
                    rec, kk):
    u_ref[:] = jnp.dot(x_ref[:], wi_ref[:], preferred_element_type=jnp.float32)
    wr = wr_ref[:]
    kf = jnp.float32(kk)

    def step(t, carry):
        a, inv_n = carry
        w = jnp.dot(a, wr, preferred_element_type=jnp.float32)
        bits = jax.lax.bitcast_convert_type(w, jnp.uint32)
        ukey = jnp.where(w < 0, ~bits, bits | jnp.uint32(0x80000000))
        prefix = _kth_threshold(ukey, kf, batch, rec)
        s = jnp.where(ukey >= prefix, w * inv_n, 0.0)
        a_new = jnp.tanh(u_ref[pl.ds(t * batch, batch), :] + s)
        nrm = jnp.sqrt(jnp.sum(a_new * a_new, axis=1, keepdims=True))
        return a_new, 1.0 / (nrm + 1e-6)

    a0 = jnp.zeros((batch, rec), jnp.float32)
    inv0 = jnp.ones((batch, 1), jnp.float32)
    a_fin, inv_fin = jax.lax.fori_loop(0, steps, step, (a0, inv0),
                                       unroll=False)
    out_ref[:] = a_fin * inv_fin


def kernel(x, W_input, W_recurrent):
    B, T, H, W, E = x.shape
    R = T * H * W
    rec = W_recurrent.shape[0]
    kk = int(rec * 0.4)
    x2 = jnp.transpose(x.reshape(B, R, E), (1, 0, 2)).reshape(R * B, E)
    return pl.pallas_call(
        functools.partial(_encoder_kernel, steps=R, batch=B, rec=rec, kk=kk),
        out_shape=jax.ShapeDtypeStruct((B, rec), x.dtype),
        scratch_shapes=[pltpu.VMEM((R * B, rec), jnp.float32)],
    )(x2, W_input, W_recurrent)


# speedup vs baseline: 1.0492x; 1.0492x over previous
import functools

import jax
import jax.numpy as jnp
from jax.experimental import pallas as pl
from jax.experimental.pallas import tpu as pltpu


def _tree(vals, op):
    while len(vals) > 1:
        nxt = [op(vals[i], vals[i + 1]) for i in range(0, len(vals) - 1, 2)]
        if len(vals) % 2:
            nxt.append(vals[-1])
        vals = nxt
    return vals[0]


def _radix_pass(ukey, prefix, cnt_at, sh, m, kf, batch):
    inds, cnts = [], []
    for j in range(1, 1 << m):
        cand = prefix | jax.lax.shift_left(jnp.uint32(j), sh)
        cnt = jnp.sum(jnp.where(ukey >= cand, 1.0, 0.0), axis=1,
                      keepdims=True)
        ok = cnt >= kf
        inds.append(jnp.where(ok, jnp.uint32(1), jnp.uint32(0)))
        cnts.append(jnp.where(ok, cnt, jnp.float32(3e9)))
    jstar = _tree(inds, lambda a, b: a + b)
    best = _tree(cnts, jnp.minimum)
    prefix = prefix | jax.lax.shift_left(jstar, sh)
    cnt_at = jnp.minimum(cnt_at, best)
    return prefix, cnt_at


def _kth_threshold(ukey, kf, batch, rec):
    prefix = jnp.zeros((batch, 1), jnp.uint32)
    cnt_at = jnp.full((batch, 1), float(rec), jnp.float32)
    prefix, cnt_at = _radix_pass(ukey, prefix, cnt_at, jnp.uint32(30), 2, kf,
                                 batch)
    prefix, cnt_at = _radix_pass(ukey, prefix, cnt_at, jnp.uint32(27), 3, kf,
                                 batch)
    prefix, cnt_at = _radix_pass(ukey, prefix, cnt_at, jnp.uint32(24), 3, kf,
                                 batch)
    prefix, cnt_at = _radix_pass(ukey, prefix, cnt_at, jnp.uint32(21), 3, kf,
                                 batch)
    prefix, cnt_at = _radix_pass(ukey, prefix, cnt_at, jnp.uint32(18), 3, kf,
                                 batch)
    prefix, cnt_at = _radix_pass(ukey, prefix, cnt_at, jnp.uint32(15), 3, kf,
                                 batch)
    prefix, cnt_at = _radix_pass(ukey, prefix, cnt_at, jnp.uint32(12), 3, kf,
                                 batch)
    def cond_fn(c):
        i, _, cnt_at = c
        return jnp.logical_and(i < 4, jnp.any(cnt_at != kf))

    def body_fn(c):
        i, prefix, cnt_at = c
        sh = jnp.uint32(9) - jnp.uint32(3) * i.astype(jnp.uint32)
        prefix, cnt_at = _radix_pass(ukey, prefix, cnt_at, sh, 3, kf, batch)
        return i + 1, prefix, cnt_at

    _, prefix, _ = jax.lax.while_loop(
        cond_fn, body_fn, (jnp.int32(0), prefix, cnt_at))
    return prefix


def _encoder_kernel(x_ref, wi_ref, wr_ref, out_ref, u_ref, *, steps, batch,
                    rec, kk):
    u_ref[:] = jnp.dot(x_ref[:], wi_ref[:], preferred_element_type=jnp.float32)
    wr = wr_ref[:]
    kf = jnp.float32(kk)

    def step(t, carry):
        a, inv_n = carry
        w = jnp.dot(a, wr, preferred_element_type=jnp.float32)
        bits = jax.lax.bitcast_convert_type(w, jnp.uint32)
        ukey = jnp.where(w < 0, ~bits, bits | jnp.uint32(0x80000000))
        prefix = _kth_threshold(ukey, kf, batch, rec)
        s = jnp.where(ukey >= prefix, w * inv_n, 0.0)
        a_new = jnp.tanh(u_ref[pl.ds(t * batch, batch), :] + s)
        nrm = jnp.sqrt(jnp.sum(a_new * a_new, axis=1, keepdims=True))
        return a_new, 1.0 / (nrm + 1e-6)

    a0 = jnp.zeros((batch, rec), jnp.float32)
    inv0 = jnp.ones((batch, 1), jnp.float32)
    a_fin, inv_fin = jax.lax.fori_loop(0, steps, step, (a0, inv0),
                                       unroll=False)
    out_ref[:] = a_fin * inv_fin


def kernel(x, W_input, W_recurrent):
    B, T, H, W, E = x.shape
    R = T * H * W
    rec = W_recurrent.shape[0]
    kk = int(rec * 0.4)
    x2 = jnp.transpose(x.reshape(B, R, E), (1, 0, 2)).reshape(R * B, E)
    return pl.pallas_call(
        functools.partial(_encoder_kernel, steps=R, batch=B, rec=rec, kk=kk),
        out_shape=jax.ShapeDtypeStruct((B, rec), x.dtype),
        scratch_shapes=[pltpu.VMEM((R * B, rec), jnp.float32)],
    )(x2, W_input, W_recurrent)
